# baseline (speedup 1.0000x reference)
import math

import jax
import jax.numpy as jnp
from jax import lax
from jax.experimental import pallas as pl
from jax.experimental.pallas import tpu as pltpu
from jax.experimental.pallas import tpu_sc as plsc

N_NODES = 10000
N_EDGES = 320000
FDIM = 128

NC = 2
NS = 16
NW = NC * NS
CHUNK = 128
K_CHUNKS = 80
EDGE_CAP = NW * K_CHUNKS * CHUNK
ACC_ROWS = 10112
ROWS_PER_TILE = ACC_ROWS // NS
BN_SCALE = 1.0 / math.sqrt(1.0 + 1e-5)
BLK = 1000

_MESH = plsc.VectorSubcoreMesh(
    core_axis_name="c", subcore_axis_name="s", num_cores=NC, num_subcores=NS)


def _make_agg(width):
  params = None if width == FDIM else pltpu.CompilerParams(
      use_tc_tiling_on_sc=False)
  nblk = 4 if width == FDIM else 1
  blkc = K_CHUNKS // nblk

  def body(src_hbm, dst_hbm, p_hbm, zeros_hbm, out_hbm,
           src_v, dst_v, rows0, rows1, acc, sem0, sem1):
    cid = lax.axis_index("c")
    sid = lax.axis_index("s")
    wid = cid * NS + sid

    z0 = sid * ROWS_PER_TILE
    pltpu.sync_copy(zeros_hbm.at[pl.ds(z0, ROWS_PER_TILE)],
                    acc.at[pl.ds(z0, ROWS_PER_TILE)])
    plsc.subcore_barrier()

    for blk in range(nblk):
      pltpu.sync_copy(src_hbm.at[wid, blk], src_v)
      pltpu.sync_copy(dst_hbm.at[wid, blk], dst_v)

      def step(i, carry):
        j0 = 2 * i
        cp0 = pltpu.async_copy(p_hbm.at[src_v.at[j0]], rows0, sem0)
        cp1 = pltpu.async_copy(p_hbm.at[src_v.at[j0 + 1]], rows1, sem1)
        cp0.wait()
        pltpu.sync_copy(rows0, acc.at[dst_v.at[j0]], add=True)
        cp1.wait()
        pltpu.sync_copy(rows1, acc.at[dst_v.at[j0 + 1]], add=True)
        return carry

      lax.fori_loop(0, blkc // 2, step, 0)

    plsc.subcore_barrier()

    pltpu.sync_copy(acc.at[pl.ds(z0, ROWS_PER_TILE)],
                    out_hbm.at[cid, pl.ds(z0, ROWS_PER_TILE)])

  return pl.kernel(
      body,
      out_type=jax.ShapeDtypeStruct((NC, ACC_ROWS, width), jnp.float32),
      mesh=_MESH,
      scratch_types=[
          pltpu.VMEM((blkc, CHUNK), jnp.int32),
          pltpu.VMEM((blkc, CHUNK), jnp.int32),
          pltpu.VMEM((CHUNK, width), jnp.float32),
          pltpu.VMEM((CHUNK, width), jnp.float32),
          pltpu.VMEM_SHARED((ACC_ROWS, width), jnp.float32),
          pltpu.SemaphoreType.DMA,
          pltpu.SemaphoreType.DMA,
      ],
      compiler_params=params)


def _count_body(dst_hbm, zeros16_hbm, ones_hbm, cnt_hbm,
                dst_v, ones_v, acc_cnt):
  cid = lax.axis_index("c")
  sid = lax.axis_index("s")
  wid = cid * NS + sid

  z0 = sid * ROWS_PER_TILE
  pltpu.sync_copy(zeros16_hbm.at[pl.ds(z0, ROWS_PER_TILE)],
                  acc_cnt.at[pl.ds(z0, ROWS_PER_TILE)])
  pltpu.sync_copy(ones_hbm, ones_v)
  pltpu.sync_copy(dst_hbm.at[wid], dst_v)
  plsc.subcore_barrier()

  def step(j, carry):
    pltpu.sync_copy(ones_v, acc_cnt.at[dst_v.at[j]], add=True)
    return carry

  lax.fori_loop(0, K_CHUNKS, step, 0)
  plsc.subcore_barrier()
  pltpu.sync_copy(acc_cnt.at[pl.ds(z0, ROWS_PER_TILE)],
                  cnt_hbm.at[cid, pl.ds(z0, ROWS_PER_TILE)])


_count = pl.kernel(
    _count_body,
    out_type=jax.ShapeDtypeStruct((NC, ACC_ROWS, 16), jnp.float32),
    mesh=_MESH,
    scratch_types=[
        pltpu.VMEM((K_CHUNKS, CHUNK), jnp.int32),
        pltpu.VMEM((CHUNK, 16), jnp.float32),
        pltpu.VMEM_SHARED((ACC_ROWS, 16), jnp.float32),
    ],
    compiler_params=pltpu.CompilerParams(use_tc_tiling_on_sc=False))


def _matmul_body(x_ref, w_ref, o_ref):
  o_ref[...] = jnp.dot(x_ref[...], w_ref[...],
                       preferred_element_type=jnp.float32)


def _matmul(x, w):
  n, d = x.shape
  return pl.pallas_call(
      _matmul_body,
      grid=(n // BLK,),
      in_specs=[
          pl.BlockSpec((BLK, d), lambda i: (i, 0)),
          pl.BlockSpec((d, w.shape[1]), lambda i: (0, 0)),
      ],
      out_specs=pl.BlockSpec((BLK, w.shape[1]), lambda i: (i, 0)),
      out_shape=jax.ShapeDtypeStruct((n, w.shape[1]), jnp.float32),
  )(x, w)


def _combine_body(part_ref, cnt_ref, x_ref, wr_ref, b_ref, g_ref, be_ref,
                  wn_ref, h_ref, p_ref):
  c = cnt_ref[0, :, 0:1] + cnt_ref[1, :, 0:1]
  inv = 1.0 / jnp.maximum(c, 1.0)
  mean = (part_ref[0] + part_ref[1]) * inv
  pre = mean + jnp.dot(x_ref[...], wr_ref[...],
                       preferred_element_type=jnp.float32) + b_ref[...]
  h = jnp.maximum(g_ref[...] * (pre * BN_SCALE) + be_ref[...], 0.0)
  h_ref[...] = h
  p_ref[...] = jnp.dot(h, wn_ref[...], preferred_element_type=jnp.float32)


def _combine(part, cnt, x, wr, b, g, be, wn):
  wnw = wn.shape[1]
  return pl.pallas_call(
      _combine_body,
      grid=(N_NODES // BLK,),
      in_specs=[
          pl.BlockSpec((2, BLK, FDIM), lambda i: (0, i, 0)),
          pl.BlockSpec((2, BLK, 16), lambda i: (0, i, 0)),
          pl.BlockSpec((BLK, FDIM), lambda i: (i, 0)),
          pl.BlockSpec((FDIM, FDIM), lambda i: (0, 0)),
          pl.BlockSpec((1, FDIM), lambda i: (0, 0)),
          pl.BlockSpec((1, FDIM), lambda i: (0, 0)),
          pl.BlockSpec((1, FDIM), lambda i: (0, 0)),
          pl.BlockSpec((FDIM, wnw), lambda i: (0, 0)),
      ],
      out_specs=[
          pl.BlockSpec((BLK, FDIM), lambda i: (i, 0)),
          pl.BlockSpec((BLK, wnw), lambda i: (i, 0)),
      ],
      out_shape=[
          jax.ShapeDtypeStruct((N_NODES, FDIM), jnp.float32),
          jax.ShapeDtypeStruct((N_NODES, wnw), jnp.float32),
      ],
  )(part, cnt, x, wr, b, g, be, wn)


def _final_body(part_ref, cnt_ref, h_ref, wr_ref, b_ref, o_ref):
  c = cnt_ref[0, :, 0:1] + cnt_ref[1, :, 0:1]
  inv = 1.0 / jnp.maximum(c, 1.0)
  mean = (part_ref[0] + part_ref[1]) * inv
  o_ref[...] = mean + jnp.dot(h_ref[...], wr_ref[...],
                              preferred_element_type=jnp.float32) + b_ref[...]


def _final(part, cnt, h, wr, b):
  return pl.pallas_call(
      _final_body,
      grid=(N_NODES // BLK,),
      in_specs=[
          pl.BlockSpec((2, BLK, 16), lambda i: (0, i, 0)),
          pl.BlockSpec((2, BLK, 16), lambda i: (0, i, 0)),
          pl.BlockSpec((BLK, FDIM), lambda i: (i, 0)),
          pl.BlockSpec((FDIM, 16), lambda i: (0, 0)),
          pl.BlockSpec((1, 16), lambda i: (0, 0)),
      ],
      out_specs=pl.BlockSpec((BLK, 16), lambda i: (i, 0)),
      out_shape=jax.ShapeDtypeStruct((N_NODES, 16), jnp.float32),
  )(part, cnt, h, wr, b)


_agg128 = _make_agg(FDIM)
_agg16 = _make_agg(16)


def kernel(x, edge_index, W_l0, b_l0, W_r0, W_l1, b_l1, W_r1,
           W_l2, b_l2, W_r2, bn0_g, bn0_b, bn1_g, bn1_b):
  pad = EDGE_CAP - N_EDGES
  src = jnp.concatenate([edge_index[0], jnp.zeros((pad,), jnp.int32)])
  src = src.reshape(NW, K_CHUNKS, CHUNK)
  dst = jnp.concatenate(
      [edge_index[1],
       N_NODES + (jnp.arange(pad, dtype=jnp.int32) % (ACC_ROWS - N_NODES))])
  dst = dst.reshape(NW, K_CHUNKS, CHUNK)

  zeros128 = jnp.zeros((ACC_ROWS, FDIM), jnp.float32)
  zeros16 = jnp.zeros((ACC_ROWS, 16), jnp.float32)
  ones16 = jnp.ones((CHUNK, 16), jnp.float32)

  b0 = b_l0.reshape(1, FDIM)
  b1 = b_l1.reshape(1, FDIM)
  g0 = bn0_g.reshape(1, FDIM)
  be0 = bn0_b.reshape(1, FDIM)
  g1 = bn1_g.reshape(1, FDIM)
  be1 = bn1_b.reshape(1, FDIM)
  wl2p = jnp.pad(W_l2, ((0, 0), (0, 16 - W_l2.shape[1])))
  wr2p = jnp.pad(W_r2, ((0, 0), (0, 16 - W_r2.shape[1])))
  b2p = jnp.pad(b_l2, (0, 16 - b_l2.shape[0])).reshape(1, 16)

  src4 = src.reshape(NW, 4, K_CHUNKS // 4, CHUNK)
  dst4 = dst.reshape(NW, 4, K_CHUNKS // 4, CHUNK)
  src1 = src.reshape(NW, 1, K_CHUNKS, CHUNK)
  dst1 = dst.reshape(NW, 1, K_CHUNKS, CHUNK)

  cnt = _count(dst, zeros16, ones16)
  p0 = _matmul(x, W_l0)
  part0 = _agg128(src4, dst4, p0, zeros128)
  h0, p1 = _combine(part0, cnt, x, W_r0, b0, g0, be0, W_l1)
  part1 = _agg128(src4, dst4, p1, zeros128)
  h1, p2 = _combine(part1, cnt, h0, W_r1, b1, g1, be1, wl2p)
  part2 = _agg16(src1, dst1, p2, zeros16)
  out = _final(part2, cnt, h1, wr2p, b2p)
  return out[:, :2]


# speedup vs baseline: 1.0384x; 1.0384x over previous
import math

import jax
import jax.numpy as jnp
from jax import lax
from jax.experimental import pallas as pl
from jax.experimental.pallas import tpu as pltpu
from jax.experimental.pallas import tpu_sc as plsc

N_NODES = 10000
N_EDGES = 320000
FDIM = 128

NC = 2
NS = 16
NW = NC * NS
CHUNK = 128
K_CHUNKS = 80
EDGE_CAP = NW * K_CHUNKS * CHUNK
ACC_ROWS = 10112
ROWS_PER_TILE = ACC_ROWS // NS
BN_SCALE = 1.0 / math.sqrt(1.0 + 1e-5)
BLK = 1000

_MESH = plsc.VectorSubcoreMesh(
    core_axis_name="c", subcore_axis_name="s", num_cores=NC, num_subcores=NS)


def _make_agg(width):
  params = None if width == FDIM else pltpu.CompilerParams(
      use_tc_tiling_on_sc=False)
  nblk = 4 if width == FDIM else 1
  blkc = K_CHUNKS // nblk

  def body(src_hbm, dst_hbm, p_hbm, zeros_hbm, out_hbm,
           src_v, dst_v, rows0, rows1, acc, sem0, sem1):
    cid = lax.axis_index("c")
    sid = lax.axis_index("s")
    wid = cid * NS + sid

    z0 = sid * ROWS_PER_TILE
    pltpu.sync_copy(zeros_hbm.at[pl.ds(z0, ROWS_PER_TILE)],
                    acc.at[pl.ds(z0, ROWS_PER_TILE)])
    plsc.subcore_barrier()

    rows = (rows0, rows1)
    sems = (sem0, sem1)
    for blk in range(nblk):
      pltpu.sync_copy(src_hbm.at[wid, blk], src_v)
      pltpu.sync_copy(dst_hbm.at[wid, blk], dst_v)

      cp = pltpu.async_copy(p_hbm.at[src_v.at[0]], rows[0], sems[0])
      for j in range(blkc):
        cp.wait()
        if j + 1 < blkc:
          nxt = (j + 1) % 2
          cp = pltpu.async_copy(p_hbm.at[src_v.at[j + 1]], rows[nxt],
                                sems[nxt])
        pltpu.sync_copy(rows[j % 2], acc.at[dst_v.at[j]], add=True)

    plsc.subcore_barrier()

    pltpu.sync_copy(acc.at[pl.ds(z0, ROWS_PER_TILE)],
                    out_hbm.at[cid, pl.ds(z0, ROWS_PER_TILE)])

  return pl.kernel(
      body,
      out_type=jax.ShapeDtypeStruct((NC, ACC_ROWS, width), jnp.float32),
      mesh=_MESH,
      scratch_types=[
          pltpu.VMEM((blkc, CHUNK), jnp.int32),
          pltpu.VMEM((blkc, CHUNK), jnp.int32),
          pltpu.VMEM((CHUNK, width), jnp.float32),
          pltpu.VMEM((CHUNK, width), jnp.float32),
          pltpu.VMEM_SHARED((ACC_ROWS, width), jnp.float32),
          pltpu.SemaphoreType.DMA,
          pltpu.SemaphoreType.DMA,
      ],
      compiler_params=params)


def _count_body(dst_hbm, zeros16_hbm, ones_hbm, cnt_hbm,
                dst_v, ones_v, acc_cnt):
  cid = lax.axis_index("c")
  sid = lax.axis_index("s")
  wid = cid * NS + sid

  z0 = sid * ROWS_PER_TILE
  pltpu.sync_copy(zeros16_hbm.at[pl.ds(z0, ROWS_PER_TILE)],
                  acc_cnt.at[pl.ds(z0, ROWS_PER_TILE)])
  pltpu.sync_copy(ones_hbm, ones_v)
  pltpu.sync_copy(dst_hbm.at[wid], dst_v)
  plsc.subcore_barrier()

  def step(j, carry):
    pltpu.sync_copy(ones_v, acc_cnt.at[dst_v.at[j]], add=True)
    return carry

  lax.fori_loop(0, K_CHUNKS, step, 0)
  plsc.subcore_barrier()
  pltpu.sync_copy(acc_cnt.at[pl.ds(z0, ROWS_PER_TILE)],
                  cnt_hbm.at[cid, pl.ds(z0, ROWS_PER_TILE)])


_count = pl.kernel(
    _count_body,
    out_type=jax.ShapeDtypeStruct((NC, ACC_ROWS, 16), jnp.float32),
    mesh=_MESH,
    scratch_types=[
        pltpu.VMEM((K_CHUNKS, CHUNK), jnp.int32),
        pltpu.VMEM((CHUNK, 16), jnp.float32),
        pltpu.VMEM_SHARED((ACC_ROWS, 16), jnp.float32),
    ],
    compiler_params=pltpu.CompilerParams(use_tc_tiling_on_sc=False))


def _matmul_body(x_ref, w_ref, o_ref):
  o_ref[...] = jnp.dot(x_ref[...], w_ref[...],
                       preferred_element_type=jnp.float32)


def _matmul(x, w):
  n, d = x.shape
  return pl.pallas_call(
      _matmul_body,
      grid=(n // BLK,),
      in_specs=[
          pl.BlockSpec((BLK, d), lambda i: (i, 0)),
          pl.BlockSpec((d, w.shape[1]), lambda i: (0, 0)),
      ],
      out_specs=pl.BlockSpec((BLK, w.shape[1]), lambda i: (i, 0)),
      out_shape=jax.ShapeDtypeStruct((n, w.shape[1]), jnp.float32),
  )(x, w)


def _combine_body(part_ref, cnt_ref, x_ref, wr_ref, b_ref, g_ref, be_ref,
                  wn_ref, h_ref, p_ref):
  c = cnt_ref[0, :, 0:1] + cnt_ref[1, :, 0:1]
  inv = 1.0 / jnp.maximum(c, 1.0)
  mean = (part_ref[0] + part_ref[1]) * inv
  pre = mean + jnp.dot(x_ref[...], wr_ref[...],
                       preferred_element_type=jnp.float32) + b_ref[...]
  h = jnp.maximum(g_ref[...] * (pre * BN_SCALE) + be_ref[...], 0.0)
  h_ref[...] = h
  p_ref[...] = jnp.dot(h, wn_ref[...], preferred_element_type=jnp.float32)


def _combine(part, cnt, x, wr, b, g, be, wn):
  wnw = wn.shape[1]
  return pl.pallas_call(
      _combine_body,
      grid=(N_NODES // BLK,),
      in_specs=[
          pl.BlockSpec((2, BLK, FDIM), lambda i: (0, i, 0)),
          pl.BlockSpec((2, BLK, 16), lambda i: (0, i, 0)),
          pl.BlockSpec((BLK, FDIM), lambda i: (i, 0)),
          pl.BlockSpec((FDIM, FDIM), lambda i: (0, 0)),
          pl.BlockSpec((1, FDIM), lambda i: (0, 0)),
          pl.BlockSpec((1, FDIM), lambda i: (0, 0)),
          pl.BlockSpec((1, FDIM), lambda i: (0, 0)),
          pl.BlockSpec((FDIM, wnw), lambda i: (0, 0)),
      ],
      out_specs=[
          pl.BlockSpec((BLK, FDIM), lambda i: (i, 0)),
          pl.BlockSpec((BLK, wnw), lambda i: (i, 0)),
      ],
      out_shape=[
          jax.ShapeDtypeStruct((N_NODES, FDIM), jnp.float32),
          jax.ShapeDtypeStruct((N_NODES, wnw), jnp.float32),
      ],
  )(part, cnt, x, wr, b, g, be, wn)


def _final_body(part_ref, cnt_ref, h_ref, wr_ref, b_ref, o_ref):
  c = cnt_ref[0, :, 0:1] + cnt_ref[1, :, 0:1]
  inv = 1.0 / jnp.maximum(c, 1.0)
  mean = (part_ref[0] + part_ref[1]) * inv
  o_ref[...] = mean + jnp.dot(h_ref[...], wr_ref[...],
                              preferred_element_type=jnp.float32) + b_ref[...]


def _final(part, cnt, h, wr, b):
  return pl.pallas_call(
      _final_body,
      grid=(N_NODES // BLK,),
      in_specs=[
          pl.BlockSpec((2, BLK, 16), lambda i: (0, i, 0)),
          pl.BlockSpec((2, BLK, 16), lambda i: (0, i, 0)),
          pl.BlockSpec((BLK, FDIM), lambda i: (i, 0)),
          pl.BlockSpec((FDIM, 16), lambda i: (0, 0)),
          pl.BlockSpec((1, 16), lambda i: (0, 0)),
      ],
      out_specs=pl.BlockSpec((BLK, 16), lambda i: (i, 0)),
      out_shape=jax.ShapeDtypeStruct((N_NODES, 16), jnp.float32),
  )(part, cnt, h, wr, b)


_agg128 = _make_agg(FDIM)
_agg16 = _make_agg(16)


def kernel(x, edge_index, W_l0, b_l0, W_r0, W_l1, b_l1, W_r1,
           W_l2, b_l2, W_r2, bn0_g, bn0_b, bn1_g, bn1_b):
  pad = EDGE_CAP - N_EDGES
  src = jnp.concatenate([edge_index[0], jnp.zeros((pad,), jnp.int32)])
  src = src.reshape(NW, K_CHUNKS, CHUNK)
  dst = jnp.concatenate(
      [edge_index[1],
       N_NODES + (jnp.arange(pad, dtype=jnp.int32) % (ACC_ROWS - N_NODES))])
  dst = dst.reshape(NW, K_CHUNKS, CHUNK)

  zeros128 = jnp.zeros((ACC_ROWS, FDIM), jnp.float32)
  zeros16 = jnp.zeros((ACC_ROWS, 16), jnp.float32)
  ones16 = jnp.ones((CHUNK, 16), jnp.float32)

  b0 = b_l0.reshape(1, FDIM)
  b1 = b_l1.reshape(1, FDIM)
  g0 = bn0_g.reshape(1, FDIM)
  be0 = bn0_b.reshape(1, FDIM)
  g1 = bn1_g.reshape(1, FDIM)
  be1 = bn1_b.reshape(1, FDIM)
  wl2p = jnp.pad(W_l2, ((0, 0), (0, 16 - W_l2.shape[1])))
  wr2p = jnp.pad(W_r2, ((0, 0), (0, 16 - W_r2.shape[1])))
  b2p = jnp.pad(b_l2, (0, 16 - b_l2.shape[0])).reshape(1, 16)

  src4 = src.reshape(NW, 4, K_CHUNKS // 4, CHUNK)
  dst4 = dst.reshape(NW, 4, K_CHUNKS // 4, CHUNK)
  src1 = src.reshape(NW, 1, K_CHUNKS, CHUNK)
  dst1 = dst.reshape(NW, 1, K_CHUNKS, CHUNK)

  cnt = _count(dst, zeros16, ones16)
  p0 = _matmul(x, W_l0)
  part0 = _agg128(src4, dst4, p0, zeros128)
  h0, p1 = _combine(part0, cnt, x, W_r0, b0, g0, be0, W_l1)
  part1 = _agg128(src4, dst4, p1, zeros128)
  h1, p2 = _combine(part1, cnt, h0, W_r1, b1, g1, be1, wl2p)
  part2 = _agg16(src1, dst1, p2, zeros16)
  out = _final(part2, cnt, h1, wr2p, b2p)
  return out[:, :2]


# speedup vs baseline: 1.9994x; 1.9255x over previous
import math

import jax
import jax.numpy as jnp
from jax import lax
from jax.experimental import pallas as pl
from jax.experimental.pallas import tpu as pltpu
from jax.experimental.pallas import tpu_sc as plsc

N_NODES = 10000
N_EDGES = 320000
FDIM = 128

NC = 2
NS = 16
NW = NC * NS
CHUNK = 128
K_CHUNKS = 79
EDGE_CAP = NW * K_CHUNKS * CHUNK
ACC_ROWS = 10112
ROWS_PER_TILE = ACC_ROWS // NS
BN_SCALE = 1.0 / math.sqrt(1.0 + 1e-5)
BLK = 1000

_MESH = plsc.VectorSubcoreMesh(
    core_axis_name="c", subcore_axis_name="s", num_cores=NC, num_subcores=NS)


def _make_agg(width):
  params = None if width == FDIM else pltpu.CompilerParams(
      use_tc_tiling_on_sc=False)

  def body(src_hbm, dst_hbm, p_hbm, zeros_hbm, out_hbm,
           src_v, dst_v, rows_v, acc, sem):
    cid = lax.axis_index("c")
    sid = lax.axis_index("s")
    wid = cid * NS + sid

    z0 = sid * ROWS_PER_TILE
    pltpu.sync_copy(zeros_hbm.at[pl.ds(z0, ROWS_PER_TILE)],
                    acc.at[pl.ds(z0, ROWS_PER_TILE)])
    pltpu.sync_copy(src_hbm.at[wid], src_v)
    pltpu.sync_copy(dst_hbm.at[wid], dst_v)
    plsc.subcore_barrier()

    def step(j, carry):
      pltpu.async_copy(p_hbm.at[src_v.at[j]], rows_v, sem).wait()
      pltpu.sync_copy(rows_v, acc.at[dst_v.at[j]], add=True)
      return carry

    lax.fori_loop(0, K_CHUNKS, step, 0)
    plsc.subcore_barrier()

    pltpu.sync_copy(acc.at[pl.ds(z0, ROWS_PER_TILE)],
                    out_hbm.at[cid, pl.ds(z0, ROWS_PER_TILE)])

  return pl.kernel(
      body,
      out_type=jax.ShapeDtypeStruct((NC, ACC_ROWS, width), jnp.float32),
      mesh=_MESH,
      scratch_types=[
          pltpu.VMEM((K_CHUNKS, CHUNK), jnp.int32),
          pltpu.VMEM((K_CHUNKS, CHUNK), jnp.int32),
          pltpu.VMEM((CHUNK, width), jnp.float32),
          pltpu.VMEM_SHARED((ACC_ROWS, width), jnp.float32),
          pltpu.SemaphoreType.DMA,
      ],
      compiler_params=params)


HALF = FDIM // 2
K2 = 160
NBLK2 = 2
BLKC2 = K2 // NBLK2
EDGE_CAP2 = NS * K2 * CHUNK


def _agg64_body(src_hbm, dst_hbm, p_hbm, zeros_hbm, out_hbm,
                src_v, dst_v, rows_v, pbuf, acc, sem):
  cid = lax.axis_index("c")
  sid = lax.axis_index("s")

  z0 = sid * ROWS_PER_TILE
  pltpu.sync_copy(zeros_hbm.at[pl.ds(z0, ROWS_PER_TILE)],
                  acc.at[pl.ds(z0, ROWS_PER_TILE)])
  pltpu.sync_copy(p_hbm.at[cid, pl.ds(z0, ROWS_PER_TILE)],
                  pbuf.at[pl.ds(z0, ROWS_PER_TILE)])
  plsc.subcore_barrier()

  for blk in range(NBLK2):
    pltpu.sync_copy(src_hbm.at[sid, blk], src_v)
    pltpu.sync_copy(dst_hbm.at[sid, blk], dst_v)

    def step(j, carry):
      pltpu.async_copy(pbuf.at[src_v.at[j]], rows_v, sem).wait()
      pltpu.sync_copy(rows_v, acc.at[dst_v.at[j]], add=True)
      return carry

    lax.fori_loop(0, BLKC2, step, 0)

  plsc.subcore_barrier()
  pltpu.sync_copy(acc.at[pl.ds(z0, ROWS_PER_TILE)],
                  out_hbm.at[cid, pl.ds(z0, ROWS_PER_TILE)])


_agg64 = pl.kernel(
    _agg64_body,
    out_type=jax.ShapeDtypeStruct((NC, ACC_ROWS, HALF), jnp.float32),
    mesh=_MESH,
    scratch_types=[
        pltpu.VMEM((BLKC2, CHUNK), jnp.int32),
        pltpu.VMEM((BLKC2, CHUNK), jnp.int32),
        pltpu.VMEM((CHUNK, HALF), jnp.float32),
        pltpu.VMEM_SHARED((ACC_ROWS, HALF), jnp.float32),
        pltpu.VMEM_SHARED((ACC_ROWS, HALF), jnp.float32),
        pltpu.SemaphoreType.DMA,
    ],
    compiler_params=pltpu.CompilerParams(use_tc_tiling_on_sc=False))


def _count_body(dst_hbm, zeros16_hbm, ones_hbm, cnt_hbm,
                dst_v, ones_v, acc_cnt):
  cid = lax.axis_index("c")
  sid = lax.axis_index("s")
  wid = cid * NS + sid

  z0 = sid * ROWS_PER_TILE
  pltpu.sync_copy(zeros16_hbm.at[pl.ds(z0, ROWS_PER_TILE)],
                  acc_cnt.at[pl.ds(z0, ROWS_PER_TILE)])
  pltpu.sync_copy(ones_hbm, ones_v)
  pltpu.sync_copy(dst_hbm.at[wid], dst_v)
  plsc.subcore_barrier()

  def step(j, carry):
    pltpu.sync_copy(ones_v, acc_cnt.at[dst_v.at[j]], add=True)
    return carry

  lax.fori_loop(0, K_CHUNKS, step, 0)
  plsc.subcore_barrier()
  pltpu.sync_copy(acc_cnt.at[pl.ds(z0, ROWS_PER_TILE)],
                  cnt_hbm.at[cid, pl.ds(z0, ROWS_PER_TILE)])


_count = pl.kernel(
    _count_body,
    out_type=jax.ShapeDtypeStruct((NC, ACC_ROWS, 16), jnp.float32),
    mesh=_MESH,
    scratch_types=[
        pltpu.VMEM((K_CHUNKS, CHUNK), jnp.int32),
        pltpu.VMEM((CHUNK, 16), jnp.float32),
        pltpu.VMEM_SHARED((ACC_ROWS, 16), jnp.float32),
    ],
    compiler_params=pltpu.CompilerParams(use_tc_tiling_on_sc=False))


def _matmul_split_body(x_ref, w_ref, o_ref):
  x = x_ref[...]
  w = w_ref[...]
  o_ref[0] = jnp.dot(x, w[:, :HALF], preferred_element_type=jnp.float32)
  o_ref[1] = jnp.dot(x, w[:, HALF:], preferred_element_type=jnp.float32)


def _matmul_split(x, w):
  return pl.pallas_call(
      _matmul_split_body,
      grid=(N_NODES // BLK,),
      in_specs=[
          pl.BlockSpec((BLK, FDIM), lambda i: (i, 0)),
          pl.BlockSpec((FDIM, FDIM), lambda i: (0, 0)),
      ],
      out_specs=pl.BlockSpec((2, BLK, HALF), lambda i: (0, i, 0)),
      out_shape=jax.ShapeDtypeStruct((2, ACC_ROWS, HALF), jnp.float32),
  )(x, w)


def _combine_split_body(part_ref, cnt_ref, x_ref, wr_ref, b_ref, g_ref,
                        be_ref, wn_ref, h_ref, p_ref):
  c = cnt_ref[0, :, 0:1] + cnt_ref[1, :, 0:1]
  inv = 1.0 / jnp.maximum(c, 1.0)
  mean = jnp.concatenate([part_ref[0], part_ref[1]], axis=-1) * inv
  pre = mean + jnp.dot(x_ref[...], wr_ref[...],
                       preferred_element_type=jnp.float32) + b_ref[...]
  h = jnp.maximum(g_ref[...] * (pre * BN_SCALE) + be_ref[...], 0.0)
  h_ref[...] = h
  p_ref[0] = jnp.dot(h, wn_ref[:, :HALF],
                     preferred_element_type=jnp.float32)
  p_ref[1] = jnp.dot(h, wn_ref[:, HALF:],
                     preferred_element_type=jnp.float32)


def _combine_flat_body(part_ref, cnt_ref, x_ref, wr_ref, b_ref, g_ref,
                       be_ref, wn_ref, h_ref, p_ref):
  c = cnt_ref[0, :, 0:1] + cnt_ref[1, :, 0:1]
  inv = 1.0 / jnp.maximum(c, 1.0)
  mean = jnp.concatenate([part_ref[0], part_ref[1]], axis=-1) * inv
  pre = mean + jnp.dot(x_ref[...], wr_ref[...],
                       preferred_element_type=jnp.float32) + b_ref[...]
  h = jnp.maximum(g_ref[...] * (pre * BN_SCALE) + be_ref[...], 0.0)
  h_ref[...] = h
  p_ref[...] = jnp.dot(h, wn_ref[...], preferred_element_type=jnp.float32)


def _combine(part, cnt, x, wr, b, g, be, wn, split):
  wnw = wn.shape[1]
  if split:
    body = _combine_split_body
    p_spec = pl.BlockSpec((2, BLK, HALF), lambda i: (0, i, 0))
    p_shape = jax.ShapeDtypeStruct((2, ACC_ROWS, HALF), jnp.float32)
  else:
    body = _combine_flat_body
    p_spec = pl.BlockSpec((BLK, wnw), lambda i: (i, 0))
    p_shape = jax.ShapeDtypeStruct((N_NODES, wnw), jnp.float32)
  return pl.pallas_call(
      body,
      grid=(N_NODES // BLK,),
      in_specs=[
          pl.BlockSpec((2, BLK, HALF), lambda i: (0, i, 0)),
          pl.BlockSpec((2, BLK, 16), lambda i: (0, i, 0)),
          pl.BlockSpec((BLK, FDIM), lambda i: (i, 0)),
          pl.BlockSpec((FDIM, FDIM), lambda i: (0, 0)),
          pl.BlockSpec((1, FDIM), lambda i: (0, 0)),
          pl.BlockSpec((1, FDIM), lambda i: (0, 0)),
          pl.BlockSpec((1, FDIM), lambda i: (0, 0)),
          pl.BlockSpec((FDIM, wnw), lambda i: (0, 0)),
      ],
      out_specs=[
          pl.BlockSpec((BLK, FDIM), lambda i: (i, 0)),
          p_spec,
      ],
      out_shape=[
          jax.ShapeDtypeStruct((N_NODES, FDIM), jnp.float32),
          p_shape,
      ],
  )(part, cnt, x, wr, b, g, be, wn)


def _final_body(part_ref, cnt_ref, h_ref, wr_ref, b_ref, o_ref):
  c = cnt_ref[0, :, 0:1] + cnt_ref[1, :, 0:1]
  inv = 1.0 / jnp.maximum(c, 1.0)
  mean = (part_ref[0] + part_ref[1]) * inv
  o_ref[...] = mean + jnp.dot(h_ref[...], wr_ref[...],
                              preferred_element_type=jnp.float32) + b_ref[...]


def _final(part, cnt, h, wr, b):
  return pl.pallas_call(
      _final_body,
      grid=(N_NODES // BLK,),
      in_specs=[
          pl.BlockSpec((2, BLK, 16), lambda i: (0, i, 0)),
          pl.BlockSpec((2, BLK, 16), lambda i: (0, i, 0)),
          pl.BlockSpec((BLK, FDIM), lambda i: (i, 0)),
          pl.BlockSpec((FDIM, 16), lambda i: (0, 0)),
          pl.BlockSpec((1, 16), lambda i: (0, 0)),
      ],
      out_specs=pl.BlockSpec((BLK, 16), lambda i: (i, 0)),
      out_shape=jax.ShapeDtypeStruct((N_NODES, 16), jnp.float32),
  )(part, cnt, h, wr, b)


_agg16 = _make_agg(16)


def kernel(x, edge_index, W_l0, b_l0, W_r0, W_l1, b_l1, W_r1,
           W_l2, b_l2, W_r2, bn0_g, bn0_b, bn1_g, bn1_b):
  pad = EDGE_CAP - N_EDGES
  src = jnp.concatenate([edge_index[0], jnp.zeros((pad,), jnp.int32)])
  src = src.reshape(NW, K_CHUNKS, CHUNK)
  dst = jnp.concatenate(
      [edge_index[1],
       N_NODES + (jnp.arange(pad, dtype=jnp.int32) % (ACC_ROWS - N_NODES))])
  dst = dst.reshape(NW, K_CHUNKS, CHUNK)

  pad2 = EDGE_CAP2 - N_EDGES
  src2 = jnp.concatenate([edge_index[0], jnp.zeros((pad2,), jnp.int32)])
  src2 = src2.reshape(NS, NBLK2, BLKC2, CHUNK)
  dst2 = jnp.concatenate(
      [edge_index[1],
       N_NODES + (jnp.arange(pad2, dtype=jnp.int32) % (ACC_ROWS - N_NODES))])
  dst2 = dst2.reshape(NS, NBLK2, BLKC2, CHUNK)

  zeros64 = jnp.zeros((ACC_ROWS, HALF), jnp.float32)
  zeros16 = jnp.zeros((ACC_ROWS, 16), jnp.float32)
  ones16 = jnp.ones((CHUNK, 16), jnp.float32)

  b0 = b_l0.reshape(1, FDIM)
  b1 = b_l1.reshape(1, FDIM)
  g0 = bn0_g.reshape(1, FDIM)
  be0 = bn0_b.reshape(1, FDIM)
  g1 = bn1_g.reshape(1, FDIM)
  be1 = bn1_b.reshape(1, FDIM)
  wl2p = jnp.pad(W_l2, ((0, 0), (0, 16 - W_l2.shape[1])))
  wr2p = jnp.pad(W_r2, ((0, 0), (0, 16 - W_r2.shape[1])))
  b2p = jnp.pad(b_l2, (0, 16 - b_l2.shape[0])).reshape(1, 16)

  cnt = _count(dst, zeros16, ones16)
  p0 = _matmul_split(x, W_l0)
  part0 = _agg64(src2, dst2, p0, zeros64)
  h0, p1 = _combine(part0, cnt, x, W_r0, b0, g0, be0, W_l1, split=True)
  part1 = _agg64(src2, dst2, p1, zeros64)
  h1, p2 = _combine(part1, cnt, h0, W_r1, b1, g1, be1, wl2p, split=False)
  part2 = _agg16(src, dst, p2, zeros16)
  out = _final(part2, cnt, h1, wr2p, b2p)
  return out[:, :2]


# speedup vs baseline: 2.1847x; 1.0927x over previous
import math

import jax
import jax.numpy as jnp
from jax import lax
from jax.experimental import pallas as pl
from jax.experimental.pallas import tpu as pltpu
from jax.experimental.pallas import tpu_sc as plsc

N_NODES = 10000
N_EDGES = 320000
FDIM = 128

NC = 2
NS = 16
NW = NC * NS
CHUNK = 128
K_CHUNKS = 79
EDGE_CAP = NW * K_CHUNKS * CHUNK
ACC_ROWS = 10112
ROWS_PER_TILE = ACC_ROWS // NS
BN_SCALE = 1.0 / math.sqrt(1.0 + 1e-5)
BLK = 1000

_MESH = plsc.VectorSubcoreMesh(
    core_axis_name="c", subcore_axis_name="s", num_cores=NC, num_subcores=NS)


def _make_agg(width):
  params = None if width == FDIM else pltpu.CompilerParams(
      use_tc_tiling_on_sc=False)

  def body(src_hbm, dst_hbm, p_hbm, zeros_hbm, out_hbm,
           src_v, dst_v, rows_v, acc, sem):
    cid = lax.axis_index("c")
    sid = lax.axis_index("s")
    wid = cid * NS + sid

    z0 = sid * ROWS_PER_TILE
    pltpu.sync_copy(zeros_hbm.at[pl.ds(z0, ROWS_PER_TILE)],
                    acc.at[pl.ds(z0, ROWS_PER_TILE)])
    pltpu.sync_copy(src_hbm.at[wid], src_v)
    pltpu.sync_copy(dst_hbm.at[wid], dst_v)
    plsc.subcore_barrier()

    def step(j, carry):
      pltpu.async_copy(p_hbm.at[src_v.at[j]], rows_v, sem).wait()
      pltpu.sync_copy(rows_v, acc.at[dst_v.at[j]], add=True)
      return carry

    lax.fori_loop(0, K_CHUNKS, step, 0)
    plsc.subcore_barrier()

    pltpu.sync_copy(acc.at[pl.ds(z0, ROWS_PER_TILE)],
                    out_hbm.at[cid, pl.ds(z0, ROWS_PER_TILE)])

  return pl.kernel(
      body,
      out_type=jax.ShapeDtypeStruct((NC, ACC_ROWS, width), jnp.float32),
      mesh=_MESH,
      scratch_types=[
          pltpu.VMEM((K_CHUNKS, CHUNK), jnp.int32),
          pltpu.VMEM((K_CHUNKS, CHUNK), jnp.int32),
          pltpu.VMEM((CHUNK, width), jnp.float32),
          pltpu.VMEM_SHARED((ACC_ROWS, width), jnp.float32),
          pltpu.SemaphoreType.DMA,
      ],
      compiler_params=params)


HALF = FDIM // 2
K2 = 160
NBLK2 = 2
BLKC2 = K2 // NBLK2
EDGE_CAP2 = NS * K2 * CHUNK


def _agg64_body(src_hbm, dst_hbm, p_hbm, zeros_hbm, out_hbm,
                src_v, dst_v, rows_v, pbuf, acc, sem):
  cid = lax.axis_index("c")
  sid = lax.axis_index("s")

  z0 = sid * ROWS_PER_TILE
  pltpu.sync_copy(zeros_hbm.at[pl.ds(z0, ROWS_PER_TILE)],
                  acc.at[pl.ds(z0, ROWS_PER_TILE)])
  pltpu.sync_copy(p_hbm.at[cid, pl.ds(z0, ROWS_PER_TILE)],
                  pbuf.at[pl.ds(z0, ROWS_PER_TILE)])
  plsc.subcore_barrier()

  for blk in range(NBLK2):
    pltpu.sync_copy(src_hbm.at[sid, blk], src_v)
    pltpu.sync_copy(dst_hbm.at[sid, blk], dst_v)

    def step(j, carry):
      pltpu.async_copy(pbuf.at[src_v.at[j]], rows_v, sem).wait()
      pltpu.sync_copy(rows_v, acc.at[dst_v.at[j]], add=True)
      return carry

    lax.fori_loop(0, BLKC2, step, 0)

  plsc.subcore_barrier()
  pltpu.sync_copy(acc.at[pl.ds(z0, ROWS_PER_TILE)],
                  out_hbm.at[cid, pl.ds(z0, ROWS_PER_TILE)])


_agg64 = pl.kernel(
    _agg64_body,
    out_type=jax.ShapeDtypeStruct((NC, ACC_ROWS, HALF), jnp.float32),
    mesh=_MESH,
    scratch_types=[
        pltpu.VMEM((BLKC2, CHUNK), jnp.int32),
        pltpu.VMEM((BLKC2, CHUNK), jnp.int32),
        pltpu.VMEM((CHUNK, HALF), jnp.float32),
        pltpu.VMEM_SHARED((ACC_ROWS, HALF), jnp.float32),
        pltpu.VMEM_SHARED((ACC_ROWS, HALF), jnp.float32),
        pltpu.SemaphoreType.DMA,
    ],
    compiler_params=pltpu.CompilerParams(use_tc_tiling_on_sc=False))


def _count_body(dst_hbm, zeros16_hbm, ones_hbm, cnt_hbm,
                dst_v, ones_v, acc_cnt):
  cid = lax.axis_index("c")
  sid = lax.axis_index("s")
  wid = cid * NS + sid

  z0 = sid * ROWS_PER_TILE
  pltpu.sync_copy(zeros16_hbm.at[pl.ds(z0, ROWS_PER_TILE)],
                  acc_cnt.at[pl.ds(z0, ROWS_PER_TILE)])
  pltpu.sync_copy(ones_hbm, ones_v)
  pltpu.sync_copy(dst_hbm.at[wid], dst_v)
  plsc.subcore_barrier()

  def step(j, carry):
    pltpu.sync_copy(ones_v, acc_cnt.at[dst_v.at[j]], add=True)
    return carry

  lax.fori_loop(0, K_CHUNKS, step, 0)
  plsc.subcore_barrier()
  pltpu.sync_copy(acc_cnt.at[pl.ds(z0, ROWS_PER_TILE)],
                  cnt_hbm.at[cid, pl.ds(z0, ROWS_PER_TILE)])


_count = pl.kernel(
    _count_body,
    out_type=jax.ShapeDtypeStruct((NC, ACC_ROWS, 16), jnp.float32),
    mesh=_MESH,
    scratch_types=[
        pltpu.VMEM((K_CHUNKS, CHUNK), jnp.int32),
        pltpu.VMEM((CHUNK, 16), jnp.float32),
        pltpu.VMEM_SHARED((ACC_ROWS, 16), jnp.float32),
    ],
    compiler_params=pltpu.CompilerParams(use_tc_tiling_on_sc=False))


def _matmul_split_body(x_ref, w_ref, o_ref):
  x = x_ref[...]
  w = w_ref[...]
  o_ref[0] = jnp.dot(x, w[:, :HALF], preferred_element_type=jnp.float32)
  o_ref[1] = jnp.dot(x, w[:, HALF:], preferred_element_type=jnp.float32)


def _matmul_split(x, w):
  return pl.pallas_call(
      _matmul_split_body,
      grid=(N_NODES // BLK,),
      in_specs=[
          pl.BlockSpec((BLK, FDIM), lambda i: (i, 0)),
          pl.BlockSpec((FDIM, FDIM), lambda i: (0, 0)),
      ],
      out_specs=pl.BlockSpec((2, BLK, HALF), lambda i: (0, i, 0)),
      out_shape=jax.ShapeDtypeStruct((2, ACC_ROWS, HALF), jnp.float32),
  )(x, w)


def _combine_split_body(part_ref, cnt_ref, x_ref, wr_ref, b_ref, g_ref,
                        be_ref, wn_ref, h_ref, p_ref):
  c = cnt_ref[0, :, 0:1] + cnt_ref[1, :, 0:1]
  inv = 1.0 / jnp.maximum(c, 1.0)
  mean = jnp.concatenate([part_ref[0], part_ref[1]], axis=-1) * inv
  pre = mean + jnp.dot(x_ref[...], wr_ref[...],
                       preferred_element_type=jnp.float32) + b_ref[...]
  h = jnp.maximum(g_ref[...] * (pre * BN_SCALE) + be_ref[...], 0.0)
  h_ref[...] = h
  p_ref[0] = jnp.dot(h, wn_ref[:, :HALF],
                     preferred_element_type=jnp.float32)
  p_ref[1] = jnp.dot(h, wn_ref[:, HALF:],
                     preferred_element_type=jnp.float32)


def _combine_flat_body(part_ref, cnt_ref, x_ref, wr_ref, b_ref, g_ref,
                       be_ref, wn_ref, h_ref, p_ref):
  c = cnt_ref[0, :, 0:1] + cnt_ref[1, :, 0:1]
  inv = 1.0 / jnp.maximum(c, 1.0)
  mean = jnp.concatenate([part_ref[0], part_ref[1]], axis=-1) * inv
  pre = mean + jnp.dot(x_ref[...], wr_ref[...],
                       preferred_element_type=jnp.float32) + b_ref[...]
  h = jnp.maximum(g_ref[...] * (pre * BN_SCALE) + be_ref[...], 0.0)
  h_ref[...] = h
  p_ref[...] = jnp.dot(h, wn_ref[...], preferred_element_type=jnp.float32)


def _combine(part, cnt, x, wr, b, g, be, wn, split):
  wnw = wn.shape[1]
  if split:
    body = _combine_split_body
    p_spec = pl.BlockSpec((2, BLK, HALF), lambda i: (0, i, 0))
    p_shape = jax.ShapeDtypeStruct((2, ACC_ROWS, HALF), jnp.float32)
  else:
    body = _combine_flat_body
    p_spec = pl.BlockSpec((BLK, wnw), lambda i: (i, 0))
    p_shape = jax.ShapeDtypeStruct((N_NODES, wnw), jnp.float32)
  return pl.pallas_call(
      body,
      grid=(N_NODES // BLK,),
      in_specs=[
          pl.BlockSpec((2, BLK, HALF), lambda i: (0, i, 0)),
          pl.BlockSpec((2, BLK, 16), lambda i: (0, i, 0)),
          pl.BlockSpec((BLK, FDIM), lambda i: (i, 0)),
          pl.BlockSpec((FDIM, FDIM), lambda i: (0, 0)),
          pl.BlockSpec((1, FDIM), lambda i: (0, 0)),
          pl.BlockSpec((1, FDIM), lambda i: (0, 0)),
          pl.BlockSpec((1, FDIM), lambda i: (0, 0)),
          pl.BlockSpec((FDIM, wnw), lambda i: (0, 0)),
      ],
      out_specs=[
          pl.BlockSpec((BLK, FDIM), lambda i: (i, 0)),
          p_spec,
      ],
      out_shape=[
          jax.ShapeDtypeStruct((N_NODES, FDIM), jnp.float32),
          p_shape,
      ],
  )(part, cnt, x, wr, b, g, be, wn)


def _final_body(part_ref, cnt_ref, h_ref, wr_ref, b_ref, o_ref):
  c = cnt_ref[0, :, 0:1] + cnt_ref[1, :, 0:1]
  inv = 1.0 / jnp.maximum(c, 1.0)
  mean = (part_ref[0] + part_ref[1]) * inv
  o_ref[...] = mean + jnp.dot(h_ref[...], wr_ref[...],
                              preferred_element_type=jnp.float32) + b_ref[...]


def _final(part, cnt, h, wr, b):
  return pl.pallas_call(
      _final_body,
      grid=(N_NODES // BLK,),
      in_specs=[
          pl.BlockSpec((2, BLK, 16), lambda i: (0, i, 0)),
          pl.BlockSpec((2, BLK, 16), lambda i: (0, i, 0)),
          pl.BlockSpec((BLK, FDIM), lambda i: (i, 0)),
          pl.BlockSpec((FDIM, 16), lambda i: (0, 0)),
          pl.BlockSpec((1, 16), lambda i: (0, 0)),
      ],
      out_specs=pl.BlockSpec((BLK, 16), lambda i: (i, 0)),
      out_shape=jax.ShapeDtypeStruct((N_NODES, 16), jnp.float32),
  )(part, cnt, h, wr, b)


def _agg16s_body(src_hbm, dst_hbm, p_hbm, zeros_hbm, out_hbm,
                 src_v, dst_v, rows_v, pbuf, acc, sem):
  cid = lax.axis_index("c")
  sid = lax.axis_index("s")
  wid = cid * NS + sid

  z0 = sid * ROWS_PER_TILE
  pltpu.sync_copy(zeros_hbm.at[pl.ds(z0, ROWS_PER_TILE)],
                  acc.at[pl.ds(z0, ROWS_PER_TILE)])
  pltpu.sync_copy(p_hbm.at[pl.ds(z0, ROWS_PER_TILE)],
                  pbuf.at[pl.ds(z0, ROWS_PER_TILE)])
  pltpu.sync_copy(src_hbm.at[wid], src_v)
  pltpu.sync_copy(dst_hbm.at[wid], dst_v)
  plsc.subcore_barrier()

  def step(j, carry):
    pltpu.async_copy(pbuf.at[src_v.at[j]], rows_v, sem).wait()
    pltpu.sync_copy(rows_v, acc.at[dst_v.at[j]], add=True)
    return carry

  lax.fori_loop(0, K_CHUNKS, step, 0)
  plsc.subcore_barrier()
  pltpu.sync_copy(acc.at[pl.ds(z0, ROWS_PER_TILE)],
                  out_hbm.at[cid, pl.ds(z0, ROWS_PER_TILE)])


_agg16s = pl.kernel(
    _agg16s_body,
    out_type=jax.ShapeDtypeStruct((NC, ACC_ROWS, 16), jnp.float32),
    mesh=_MESH,
    scratch_types=[
        pltpu.VMEM((K_CHUNKS, CHUNK), jnp.int32),
        pltpu.VMEM((K_CHUNKS, CHUNK), jnp.int32),
        pltpu.VMEM((CHUNK, 16), jnp.float32),
        pltpu.VMEM_SHARED((ACC_ROWS, 16), jnp.float32),
        pltpu.VMEM_SHARED((ACC_ROWS, 16), jnp.float32),
        pltpu.SemaphoreType.DMA,
    ],
    compiler_params=pltpu.CompilerParams(use_tc_tiling_on_sc=False))


def kernel(x, edge_index, W_l0, b_l0, W_r0, W_l1, b_l1, W_r1,
           W_l2, b_l2, W_r2, bn0_g, bn0_b, bn1_g, bn1_b):
  pad = EDGE_CAP - N_EDGES
  src = jnp.concatenate([edge_index[0], jnp.zeros((pad,), jnp.int32)])
  src = src.reshape(NW, K_CHUNKS, CHUNK)
  dst = jnp.concatenate(
      [edge_index[1],
       N_NODES + (jnp.arange(pad, dtype=jnp.int32) % (ACC_ROWS - N_NODES))])
  dst = dst.reshape(NW, K_CHUNKS, CHUNK)

  pad2 = EDGE_CAP2 - N_EDGES
  src2 = jnp.concatenate([edge_index[0], jnp.zeros((pad2,), jnp.int32)])
  src2 = src2.reshape(NS, NBLK2, BLKC2, CHUNK)
  dst2 = jnp.concatenate(
      [edge_index[1],
       N_NODES + (jnp.arange(pad2, dtype=jnp.int32) % (ACC_ROWS - N_NODES))])
  dst2 = dst2.reshape(NS, NBLK2, BLKC2, CHUNK)

  zeros64 = jnp.zeros((ACC_ROWS, HALF), jnp.float32)
  zeros16 = jnp.zeros((ACC_ROWS, 16), jnp.float32)
  ones16 = jnp.ones((CHUNK, 16), jnp.float32)

  b0 = b_l0.reshape(1, FDIM)
  b1 = b_l1.reshape(1, FDIM)
  g0 = bn0_g.reshape(1, FDIM)
  be0 = bn0_b.reshape(1, FDIM)
  g1 = bn1_g.reshape(1, FDIM)
  be1 = bn1_b.reshape(1, FDIM)
  wl2p = jnp.pad(W_l2, ((0, 0), (0, 16 - W_l2.shape[1])))
  wr2p = jnp.pad(W_r2, ((0, 0), (0, 16 - W_r2.shape[1])))
  b2p = jnp.pad(b_l2, (0, 16 - b_l2.shape[0])).reshape(1, 16)

  cnt = _count(dst, zeros16, ones16)
  p0 = _matmul_split(x, W_l0)
  part0 = _agg64(src2, dst2, p0, zeros64)
  h0, p1 = _combine(part0, cnt, x, W_r0, b0, g0, be0, W_l1, split=True)
  part1 = _agg64(src2, dst2, p1, zeros64)
  h1, p2 = _combine(part1, cnt, h0, W_r1, b1, g1, be1, wl2p, split=False)
  p2pad = jnp.pad(p2, ((0, ACC_ROWS - N_NODES), (0, 0)))
  part2 = _agg16s(src, dst, p2pad, zeros16)
  out = _final(part2, cnt, h1, wr2p, b2p)
  return out[:, :2]
